# Initial kernel scaffold; baseline (speedup 1.0000x reference)
import jax
import jax.numpy as jnp
from jax.experimental import pallas as pl


def kernel(x, edge_index, W, b, Wih1, Whh1, bih1, bhh1, Wih2, Whh2, bih2, bhh2, Wc, bc):
    raise NotImplementedError("write your pallas kernel here")


# speedup vs baseline: 12.1757x; 12.1757x over previous
import functools

import jax
import jax.numpy as jnp
from jax import lax
from jax.experimental import pallas as pl
from jax.experimental.pallas import tpu as pltpu
from jax.experimental.pallas import tpu_sc as plsc

_N, _E, _D, _H, _O = 10000, 320000, 128, 128, 16
_NP = 10240
_NC, _NS = 2, 16
_EPW = _E // (_NC * _NS)
_CH = 80

_mesh = plsc.VectorSubcoreMesh(core_axis_name="c", subcore_axis_name="s")


def _deg_body(dst_hbm, out_hbm, ones_v, idx_v, zbuf, deg_sh, sem):
    c = lax.axis_index("c")
    s = lax.axis_index("s")

    @pl.loop(0, _CH, step=16)
    def _(i):
        ones_v[pl.ds(i, 16)] = jnp.ones((16,), jnp.float32)

    @pl.loop(0, 640, step=16)
    def _(i):
        zbuf[pl.ds(i, 16)] = jnp.zeros((16,), jnp.float32)

    pltpu.sync_copy(zbuf, deg_sh.at[pl.ds(s * 640, 640)])
    plsc.subcore_barrier()

    base = (c * _NS + s) * _EPW

    @pl.loop(0, _EPW, step=_CH)
    def _(j):
        pltpu.sync_copy(dst_hbm.at[pl.ds(base + j, _CH)], idx_v)
        pltpu.sync_copy(ones_v, deg_sh.at[idx_v], add=True)

    plsc.subcore_barrier()
    pltpu.sync_copy(deg_sh.at[pl.ds(s * 640, 640)],
                    out_hbm.at[c, pl.ds(s * 640, 640)])


def _sc_degree(dst):
    k = pl.kernel(
        _deg_body,
        out_type=jax.ShapeDtypeStruct((_NC, _NP), jnp.float32),
        mesh=_mesh,
        scratch_types=[
            pltpu.VMEM((_CH,), jnp.float32),
            pltpu.VMEM((_CH,), jnp.int32),
            pltpu.VMEM((640,), jnp.float32),
            pltpu.VMEM_SHARED((_NP,), jnp.float32),
            pltpu.SemaphoreType.DMA,
        ],
    )
    return k(dst)


def _msg_body(xwn_hbm, src_hbm, dst_hbm, out_hbm,
              src_v, dst_v, rows_v, zbuf, acc_sh, sem):
    c = lax.axis_index("c")
    s = lax.axis_index("s")

    @pl.loop(0, 128)
    def _(i):
        @pl.loop(0, 128, step=16)
        def _(j):
            zbuf[i, pl.ds(j, 16)] = jnp.zeros((16,), jnp.float32)

    @pl.loop(0, 5)
    def _(kk):
        pltpu.sync_copy(zbuf, acc_sh.at[pl.ds(s * 640 + kk * 128, 128), :])

    plsc.subcore_barrier()

    base = (c * _NS + s) * _EPW

    @pl.loop(0, _EPW, step=_CH)
    def _(j):
        pltpu.sync_copy(src_hbm.at[pl.ds(base + j, _CH)], src_v)
        pltpu.sync_copy(dst_hbm.at[pl.ds(base + j, _CH)], dst_v)
        pltpu.async_copy(xwn_hbm.at[src_v], rows_v, sem).wait()
        pltpu.sync_copy(rows_v, acc_sh.at[dst_v], add=True)

    plsc.subcore_barrier()

    @pl.loop(0, 5)
    def _(kk):
        pltpu.sync_copy(acc_sh.at[pl.ds(s * 640 + kk * 128, 128), :],
                        out_hbm.at[c, pl.ds(s * 640 + kk * 128, 128), :])


def _sc_message(xwn, src, dst):
    k = pl.kernel(
        _msg_body,
        out_type=jax.ShapeDtypeStruct((_NC, _NP, _H), jnp.float32),
        mesh=_mesh,
        scratch_types=[
            pltpu.VMEM((_CH,), jnp.int32),
            pltpu.VMEM((_CH,), jnp.int32),
            pltpu.VMEM((_CH, _H), jnp.float32),
            pltpu.VMEM((128, _H), jnp.float32),
            pltpu.VMEM_SHARED((_NP, _H), jnp.float32),
            pltpu.SemaphoreType.DMA,
        ],
    )
    return k(xwn, src, dst)


def _xw_body(x_ref, w_ref, o_ref):
    o_ref[...] = jnp.dot(x_ref[...], w_ref[...],
                         preferred_element_type=jnp.float32)


def _tc_xw(x, W):
    return pl.pallas_call(
        _xw_body,
        out_shape=jax.ShapeDtypeStruct((_N, _H), jnp.float32),
        grid=(5,),
        in_specs=[pl.BlockSpec((2000, _D), lambda i: (i, 0)),
                  pl.BlockSpec((_D, _H), lambda i: (0, 0))],
        out_specs=pl.BlockSpec((2000, _H), lambda i: (i, 0)),
    )(x, W)


def _scale_body(p0_ref, p1_ref, xw_ref, xwn_ref, dinv_ref):
    deg = 1.0 + p0_ref[...] + p1_ref[...]
    dinv = lax.rsqrt(jnp.maximum(deg, 1.0))
    dinv_ref[...] = dinv
    xwn_ref[...] = xw_ref[...] * dinv


def _tc_scale(p0, p1, xw):
    blk = 1000
    return pl.pallas_call(
        _scale_body,
        out_shape=(jax.ShapeDtypeStruct((_N, _H), jnp.float32),
                   jax.ShapeDtypeStruct((_N, 1), jnp.float32)),
        grid=(_N // blk,),
        in_specs=[pl.BlockSpec((blk, 1), lambda i: (i, 0)),
                  pl.BlockSpec((blk, 1), lambda i: (i, 0)),
                  pl.BlockSpec((blk, _H), lambda i: (i, 0))],
        out_specs=(pl.BlockSpec((blk, _H), lambda i: (i, 0)),
                   pl.BlockSpec((blk, 1), lambda i: (i, 0))),
    )(p0, p1, xw)


def _fin_body(p0_ref, p1_ref, xwn_ref, dinv_ref, b_ref, h0_ref):
    h0_ref[...] = jnp.maximum(
        dinv_ref[...] * (p0_ref[...] + p1_ref[...] + xwn_ref[...])
        + b_ref[...], 0.0)


def _tc_finalize(p0, p1, xwn, dinv, b):
    blk = 1000
    return pl.pallas_call(
        _fin_body,
        out_shape=jax.ShapeDtypeStruct((_N, _H), jnp.float32),
        grid=(_N // blk,),
        in_specs=[pl.BlockSpec((blk, _H), lambda i: (i, 0)),
                  pl.BlockSpec((blk, _H), lambda i: (i, 0)),
                  pl.BlockSpec((blk, _H), lambda i: (i, 0)),
                  pl.BlockSpec((blk, 1), lambda i: (i, 0)),
                  pl.BlockSpec((1, _H), lambda i: (0, 0))],
        out_specs=pl.BlockSpec((blk, _H), lambda i: (i, 0)),
    )(p0, p1, xwn, dinv, b)


_MBLK = 1000


def _blocked_proj(src_ref, wt_ref, bias, dst_ref):
    def body(i, _):
        base = pl.multiple_of(i * _MBLK, 8)
        rows = src_ref[pl.ds(base, _MBLK), :]
        dst_ref[pl.ds(base, _MBLK), :] = jnp.dot(
            rows, wt_ref[...], preferred_element_type=jnp.float32) + bias
        return 0
    lax.fori_loop(0, _N // _MBLK, body, 0)


def _gru_body(h0_ref,
              wih1t_ref, whh1t_ref, bih1_ref, bhh1_ref,
              wih2t_ref, whh2t_ref, bih2_ref, bhh2_ref,
              wc_ref, bc_ref, out_ref, gi_ref, ys1_ref, ys2_ref):
    _blocked_proj(h0_ref, wih1t_ref, bih1_ref[...], gi_ref)

    def make_step(whht_ref, bhh_ref, dst_ref):
        bhh = bhh_ref[...]

        def step(t, h):
            gh = jnp.dot(h, whht_ref[...],
                         preferred_element_type=jnp.float32) + bhh
            gi = gi_ref[pl.ds(t, 1), :]
            r = jax.nn.sigmoid(gi[:, :_H] + gh[:, :_H])
            z = jax.nn.sigmoid(gi[:, _H:2 * _H] + gh[:, _H:2 * _H])
            n = jnp.tanh(gi[:, 2 * _H:] + r * gh[:, 2 * _H:])
            h2 = (1.0 - z) * n + z * h
            dst_ref[pl.ds(t, 1), :] = h2
            return h2
        return step

    h_init = jnp.zeros((1, _H), jnp.float32)
    lax.fori_loop(0, _N, make_step(whh1t_ref, bhh1_ref, ys1_ref), h_init)

    _blocked_proj(ys1_ref, wih2t_ref, bih2_ref[...], gi_ref)
    lax.fori_loop(0, _N, make_step(whh2t_ref, bhh2_ref, ys2_ref), h_init)

    bc = bc_ref[...]

    def out_body(i, _):
        base = pl.multiple_of(i * _MBLK, 8)
        rows = ys2_ref[pl.ds(base, _MBLK), :]
        out_ref[pl.ds(base, _MBLK), :] = jnp.dot(
            rows, wc_ref[...], preferred_element_type=jnp.float32) + bc
        return 0
    lax.fori_loop(0, _N // _MBLK, out_body, 0)


def _tc_gru(h0, wih1t, whh1t, bih1, bhh1,
            wih2t, whh2t, bih2, bhh2, wc, bc):
    return pl.pallas_call(
        _gru_body,
        out_shape=jax.ShapeDtypeStruct((_N, _O), jnp.float32),
        scratch_shapes=[
            pltpu.VMEM((_N, 3 * _H), jnp.float32),
            pltpu.VMEM((_N, _H), jnp.float32),
            pltpu.VMEM((_N, _H), jnp.float32),
        ],
    )(h0, wih1t, whh1t, bih1, bhh1, wih2t, whh2t, bih2, bhh2, wc, bc)


def kernel(x, edge_index, W, b, Wih1, Whh1, bih1, bhh1,
           Wih2, Whh2, bih2, bhh2, Wc, bc):
    src = edge_index[0]
    dst = edge_index[1]

    deg_p = _sc_degree(dst)
    xw = _tc_xw(x, W)

    degp0 = deg_p[0, :_N].reshape(_N, 1)
    degp1 = deg_p[1, :_N].reshape(_N, 1)
    xwn, dinv = _tc_scale(degp0, degp1, xw)

    acc_p = _sc_message(xwn, src, dst)
    p0 = acc_p[0, :_N]
    p1 = acc_p[1, :_N]

    h0 = _tc_finalize(p0, p1, xwn, dinv, b.reshape(1, _H))
    out = _tc_gru(
        h0,
        Wih1.T, Whh1.T, bih1.reshape(1, 3 * _H), bhh1.reshape(1, 3 * _H),
        Wih2.T, Whh2.T, bih2.reshape(1, 3 * _H), bhh2.reshape(1, 3 * _H),
        Wc, bc.reshape(1, _O))
    return out


# speedup vs baseline: 16.8561x; 1.3844x over previous
import functools

import jax
import jax.numpy as jnp
from jax import lax
from jax.experimental import pallas as pl
from jax.experimental.pallas import tpu as pltpu
from jax.experimental.pallas import tpu_sc as plsc

_N, _E, _D, _H, _O = 10000, 320000, 128, 128, 16
_NP = 10240
_NC, _NS = 2, 16
_EPW = _E // (_NC * _NS)
_CH = 80

_mesh = plsc.VectorSubcoreMesh(core_axis_name="c", subcore_axis_name="s")


def _deg_body(dst_hbm, out_hbm, ones_v, idx_v, zbuf, deg_sh, sem):
    c = lax.axis_index("c")
    s = lax.axis_index("s")

    @pl.loop(0, _CH, step=16)
    def _(i):
        ones_v[pl.ds(i, 16)] = jnp.ones((16,), jnp.float32)

    @pl.loop(0, 640, step=16)
    def _(i):
        zbuf[pl.ds(i, 16)] = jnp.zeros((16,), jnp.float32)

    pltpu.sync_copy(zbuf, deg_sh.at[pl.ds(s * 640, 640)])
    plsc.subcore_barrier()

    base = (c * _NS + s) * _EPW

    @pl.loop(0, _EPW, step=_CH)
    def _(j):
        pltpu.sync_copy(dst_hbm.at[pl.ds(base + j, _CH)], idx_v)
        pltpu.sync_copy(ones_v, deg_sh.at[idx_v], add=True)

    plsc.subcore_barrier()
    pltpu.sync_copy(deg_sh.at[pl.ds(s * 640, 640)],
                    out_hbm.at[c, pl.ds(s * 640, 640)])


def _sc_degree(dst):
    k = pl.kernel(
        _deg_body,
        out_type=jax.ShapeDtypeStruct((_NC, _NP), jnp.float32),
        mesh=_mesh,
        scratch_types=[
            pltpu.VMEM((_CH,), jnp.float32),
            pltpu.VMEM((_CH,), jnp.int32),
            pltpu.VMEM((640,), jnp.float32),
            pltpu.VMEM_SHARED((_NP,), jnp.float32),
            pltpu.SemaphoreType.DMA,
        ],
    )
    return k(dst)


def _msg_body(xwn_hbm, src_hbm, dst_hbm, out_hbm,
              src_v, dst_v, rows_v, zbuf, acc_sh, sem):
    c = lax.axis_index("c")
    s = lax.axis_index("s")

    @pl.loop(0, 128)
    def _(i):
        @pl.loop(0, 128, step=16)
        def _(j):
            zbuf[i, pl.ds(j, 16)] = jnp.zeros((16,), jnp.float32)

    @pl.loop(0, 5)
    def _(kk):
        pltpu.sync_copy(zbuf, acc_sh.at[pl.ds(s * 640 + kk * 128, 128), :])

    plsc.subcore_barrier()

    base = (c * _NS + s) * _EPW

    @pl.loop(0, _EPW, step=_CH)
    def _(j):
        pltpu.sync_copy(src_hbm.at[pl.ds(base + j, _CH)], src_v)
        pltpu.sync_copy(dst_hbm.at[pl.ds(base + j, _CH)], dst_v)
        pltpu.async_copy(xwn_hbm.at[src_v], rows_v, sem).wait()
        pltpu.sync_copy(rows_v, acc_sh.at[dst_v], add=True)

    plsc.subcore_barrier()

    @pl.loop(0, 5)
    def _(kk):
        pltpu.sync_copy(acc_sh.at[pl.ds(s * 640 + kk * 128, 128), :],
                        out_hbm.at[c, pl.ds(s * 640 + kk * 128, 128), :])


def _sc_message(xwn, src, dst):
    k = pl.kernel(
        _msg_body,
        out_type=jax.ShapeDtypeStruct((_NC, _NP, _H), jnp.float32),
        mesh=_mesh,
        scratch_types=[
            pltpu.VMEM((_CH,), jnp.int32),
            pltpu.VMEM((_CH,), jnp.int32),
            pltpu.VMEM((_CH, _H), jnp.float32),
            pltpu.VMEM((128, _H), jnp.float32),
            pltpu.VMEM_SHARED((_NP, _H), jnp.float32),
            pltpu.SemaphoreType.DMA,
        ],
    )
    return k(xwn, src, dst)


def _xw_body(x_ref, w_ref, o_ref):
    o_ref[...] = jnp.dot(x_ref[...], w_ref[...],
                         preferred_element_type=jnp.float32)


def _tc_xw(x, W):
    return pl.pallas_call(
        _xw_body,
        out_shape=jax.ShapeDtypeStruct((_N, _H), jnp.float32),
        grid=(5,),
        in_specs=[pl.BlockSpec((2000, _D), lambda i: (i, 0)),
                  pl.BlockSpec((_D, _H), lambda i: (0, 0))],
        out_specs=pl.BlockSpec((2000, _H), lambda i: (i, 0)),
    )(x, W)


def _scale_body(p0_ref, p1_ref, xw_ref, xwn_ref, dinv_ref):
    deg = 1.0 + p0_ref[...] + p1_ref[...]
    dinv = lax.rsqrt(jnp.maximum(deg, 1.0))
    dinv_ref[...] = dinv
    xwn_ref[...] = xw_ref[...] * dinv


def _tc_scale(p0, p1, xw):
    blk = 1000
    return pl.pallas_call(
        _scale_body,
        out_shape=(jax.ShapeDtypeStruct((_N, _H), jnp.float32),
                   jax.ShapeDtypeStruct((_N, 1), jnp.float32)),
        grid=(_N // blk,),
        in_specs=[pl.BlockSpec((blk, 1), lambda i: (i, 0)),
                  pl.BlockSpec((blk, 1), lambda i: (i, 0)),
                  pl.BlockSpec((blk, _H), lambda i: (i, 0))],
        out_specs=(pl.BlockSpec((blk, _H), lambda i: (i, 0)),
                   pl.BlockSpec((blk, 1), lambda i: (i, 0))),
    )(p0, p1, xw)


def _fin_body(p0_ref, p1_ref, xwn_ref, dinv_ref, b_ref, h0_ref):
    h0_ref[...] = jnp.maximum(
        dinv_ref[...] * (p0_ref[...] + p1_ref[...] + xwn_ref[...])
        + b_ref[...], 0.0)


def _tc_finalize(p0, p1, xwn, dinv, b):
    blk = 1000
    return pl.pallas_call(
        _fin_body,
        out_shape=jax.ShapeDtypeStruct((_N, _H), jnp.float32),
        grid=(_N // blk,),
        in_specs=[pl.BlockSpec((blk, _H), lambda i: (i, 0)),
                  pl.BlockSpec((blk, _H), lambda i: (i, 0)),
                  pl.BlockSpec((blk, _H), lambda i: (i, 0)),
                  pl.BlockSpec((blk, 1), lambda i: (i, 0)),
                  pl.BlockSpec((1, _H), lambda i: (0, 0))],
        out_specs=pl.BlockSpec((blk, _H), lambda i: (i, 0)),
    )(p0, p1, xwn, dinv, b)


_MBLK = 1000


def _blocked_proj(src_ref, wt_ref, bias, dst_ref):
    def body(i, _):
        base = pl.multiple_of(i * _MBLK, 8)
        rows = src_ref[pl.ds(base, _MBLK), :]
        dst_ref[pl.ds(base, _MBLK), :] = jnp.dot(
            rows, wt_ref[...], preferred_element_type=jnp.float32) + bias
        return 0
    lax.fori_loop(0, _N // _MBLK, body, 0)


def _gru_body(h0_ref, wih1t_ref, bih1_ref, bigw_ref, bb_ref,
              wc_ref, bc_ref, out_ref, gi_ref, ys2_ref):
    _blocked_proj(h0_ref, wih1t_ref, bih1_ref[...], gi_ref)

    bb = bb_ref[...]

    def step(t, carry):
        h1, h2 = carry
        hcat = jnp.concatenate([h1, h2], axis=1)
        m = jnp.dot(hcat, bigw_ref[...],
                    preferred_element_type=jnp.float32) + bb
        gh1 = m[:, :3 * _H]
        g2 = m[:, 3 * _H:6 * _H]
        gh2 = m[:, 6 * _H:]
        tt = jnp.minimum(t, _N - 1)
        gi1 = gi_ref[pl.ds(tt, 1), :]
        r1 = jax.nn.sigmoid(gi1[:, :_H] + gh1[:, :_H])
        z1 = jax.nn.sigmoid(gi1[:, _H:2 * _H] + gh1[:, _H:2 * _H])
        n1 = jnp.tanh(gi1[:, 2 * _H:] + r1 * gh1[:, 2 * _H:])
        h1n = (1.0 - z1) * n1 + z1 * h1
        r2 = jax.nn.sigmoid(g2[:, :_H] + gh2[:, :_H])
        z2 = jax.nn.sigmoid(g2[:, _H:2 * _H] + gh2[:, _H:2 * _H])
        n2 = jnp.tanh(g2[:, 2 * _H:] + r2 * gh2[:, 2 * _H:])
        h2n = (1.0 - z2) * n2 + z2 * h2

        @pl.when(t > 0)
        def _():
            ys2_ref[pl.ds(t - 1, 1), :] = h2n

        h2n = jnp.where(t > 0, h2n, 0.0)
        return (h1n, h2n)

    h_init = jnp.zeros((1, _H), jnp.float32)
    lax.fori_loop(0, _N + 1, step, (h_init, h_init))

    bc = bc_ref[...]

    def out_body(i, _):
        base = pl.multiple_of(i * _MBLK, 8)
        rows = ys2_ref[pl.ds(base, _MBLK), :]
        out_ref[pl.ds(base, _MBLK), :] = jnp.dot(
            rows, wc_ref[...], preferred_element_type=jnp.float32) + bc
        return 0
    lax.fori_loop(0, _N // _MBLK, out_body, 0)


def _tc_gru(h0, wih1t, bih1, bigw, bb, wc, bc):
    return pl.pallas_call(
        _gru_body,
        out_shape=jax.ShapeDtypeStruct((_N, _O), jnp.float32),
        scratch_shapes=[
            pltpu.VMEM((_N, 3 * _H), jnp.float32),
            pltpu.VMEM((_N, _H), jnp.float32),
        ],
    )(h0, wih1t, bih1, bigw, bb, wc, bc)


def kernel(x, edge_index, W, b, Wih1, Whh1, bih1, bhh1,
           Wih2, Whh2, bih2, bhh2, Wc, bc):
    src = edge_index[0]
    dst = edge_index[1]

    deg_p = _sc_degree(dst)
    xw = _tc_xw(x, W)

    degp0 = deg_p[0, :_N].reshape(_N, 1)
    degp1 = deg_p[1, :_N].reshape(_N, 1)
    xwn, dinv = _tc_scale(degp0, degp1, xw)

    acc_p = _sc_message(xwn, src, dst)
    p0 = acc_p[0, :_N]
    p1 = acc_p[1, :_N]

    h0 = _tc_finalize(p0, p1, xwn, dinv, b.reshape(1, _H))
    bigw = jnp.zeros((2 * _H, 9 * _H), jnp.float32)
    bigw = bigw.at[:_H, :3 * _H].set(Whh1.T)
    bigw = bigw.at[:_H, 3 * _H:6 * _H].set(Wih2.T)
    bigw = bigw.at[_H:, 6 * _H:].set(Whh2.T)
    bb = jnp.concatenate([bhh1, bih2, bhh2]).reshape(1, 9 * _H)
    out = _tc_gru(
        h0, Wih1.T, bih1.reshape(1, 3 * _H), bigw, bb,
        Wc, bc.reshape(1, _O))
    return out


# speedup vs baseline: 19.5768x; 1.1614x over previous
import functools

import jax
import jax.numpy as jnp
from jax import lax
from jax.experimental import pallas as pl
from jax.experimental.pallas import tpu as pltpu
from jax.experimental.pallas import tpu_sc as plsc

_N, _E, _D, _H, _O = 10000, 320000, 128, 128, 16
_NP = 10240
_NC, _NS = 2, 16
_EPW = _E // (_NC * _NS)
_CH = 80

_mesh = plsc.VectorSubcoreMesh(core_axis_name="c", subcore_axis_name="s")


def _deg_body(dst_hbm, out_hbm, ones_v, idx_v, zbuf, deg_sh, sem):
    c = lax.axis_index("c")
    s = lax.axis_index("s")

    @pl.loop(0, _CH, step=16)
    def _(i):
        ones_v[pl.ds(i, 16)] = jnp.ones((16,), jnp.float32)

    @pl.loop(0, 640, step=16)
    def _(i):
        zbuf[pl.ds(i, 16)] = jnp.zeros((16,), jnp.float32)

    pltpu.sync_copy(zbuf, deg_sh.at[pl.ds(s * 640, 640)])
    plsc.subcore_barrier()

    base = (c * _NS + s) * _EPW

    @pl.loop(0, _EPW, step=_CH)
    def _(j):
        pltpu.sync_copy(dst_hbm.at[pl.ds(base + j, _CH)], idx_v)
        pltpu.sync_copy(ones_v, deg_sh.at[idx_v], add=True)

    plsc.subcore_barrier()
    pltpu.sync_copy(deg_sh.at[pl.ds(s * 640, 640)],
                    out_hbm.at[c, pl.ds(s * 640, 640)])


def _sc_degree(dst):
    k = pl.kernel(
        _deg_body,
        out_type=jax.ShapeDtypeStruct((_NC, _NP), jnp.float32),
        mesh=_mesh,
        scratch_types=[
            pltpu.VMEM((_CH,), jnp.float32),
            pltpu.VMEM((_CH,), jnp.int32),
            pltpu.VMEM((640,), jnp.float32),
            pltpu.VMEM_SHARED((_NP,), jnp.float32),
            pltpu.SemaphoreType.DMA,
        ],
    )
    return k(dst)


def _msg_body(xwn_hbm, src_hbm, dst_hbm, out_hbm,
              src_v, dst_v, rows_v, zbuf, acc_sh, sem):
    c = lax.axis_index("c")
    s = lax.axis_index("s")

    @pl.loop(0, 128)
    def _(i):
        @pl.loop(0, 128, step=16)
        def _(j):
            zbuf[i, pl.ds(j, 16)] = jnp.zeros((16,), jnp.float32)

    @pl.loop(0, 5)
    def _(kk):
        pltpu.sync_copy(zbuf, acc_sh.at[pl.ds(s * 640 + kk * 128, 128), :])

    plsc.subcore_barrier()

    base = (c * _NS + s) * _EPW

    @pl.loop(0, _EPW, step=_CH)
    def _(j):
        pltpu.sync_copy(src_hbm.at[pl.ds(base + j, _CH)], src_v)
        pltpu.sync_copy(dst_hbm.at[pl.ds(base + j, _CH)], dst_v)
        pltpu.async_copy(xwn_hbm.at[src_v], rows_v, sem).wait()
        pltpu.sync_copy(rows_v, acc_sh.at[dst_v], add=True)

    plsc.subcore_barrier()

    @pl.loop(0, 5)
    def _(kk):
        pltpu.sync_copy(acc_sh.at[pl.ds(s * 640 + kk * 128, 128), :],
                        out_hbm.at[c, pl.ds(s * 640 + kk * 128, 128), :])


def _sc_message(xwn, src, dst):
    k = pl.kernel(
        _msg_body,
        out_type=jax.ShapeDtypeStruct((_NC, _NP, _H), jnp.float32),
        mesh=_mesh,
        scratch_types=[
            pltpu.VMEM((_CH,), jnp.int32),
            pltpu.VMEM((_CH,), jnp.int32),
            pltpu.VMEM((_CH, _H), jnp.float32),
            pltpu.VMEM((128, _H), jnp.float32),
            pltpu.VMEM_SHARED((_NP, _H), jnp.float32),
            pltpu.SemaphoreType.DMA,
        ],
    )
    return k(xwn, src, dst)


def _xw_body(x_ref, w_ref, o_ref):
    o_ref[...] = jnp.dot(x_ref[...], w_ref[...],
                         preferred_element_type=jnp.float32)


def _tc_xw(x, W):
    return pl.pallas_call(
        _xw_body,
        out_shape=jax.ShapeDtypeStruct((_N, _H), jnp.float32),
        grid=(5,),
        in_specs=[pl.BlockSpec((2000, _D), lambda i: (i, 0)),
                  pl.BlockSpec((_D, _H), lambda i: (0, 0))],
        out_specs=pl.BlockSpec((2000, _H), lambda i: (i, 0)),
    )(x, W)


def _scale_body(p0_ref, p1_ref, xw_ref, xwn_ref, dinv_ref):
    deg = 1.0 + p0_ref[...] + p1_ref[...]
    dinv = lax.rsqrt(jnp.maximum(deg, 1.0))
    dinv_ref[...] = dinv
    xwn_ref[...] = xw_ref[...] * dinv


def _tc_scale(p0, p1, xw):
    blk = 1000
    return pl.pallas_call(
        _scale_body,
        out_shape=(jax.ShapeDtypeStruct((_N, _H), jnp.float32),
                   jax.ShapeDtypeStruct((_N, 1), jnp.float32)),
        grid=(_N // blk,),
        in_specs=[pl.BlockSpec((blk, 1), lambda i: (i, 0)),
                  pl.BlockSpec((blk, 1), lambda i: (i, 0)),
                  pl.BlockSpec((blk, _H), lambda i: (i, 0))],
        out_specs=(pl.BlockSpec((blk, _H), lambda i: (i, 0)),
                   pl.BlockSpec((blk, 1), lambda i: (i, 0))),
    )(p0, p1, xw)


def _fin_body(p0_ref, p1_ref, xwn_ref, dinv_ref, b_ref, h0_ref):
    h0_ref[...] = jnp.maximum(
        dinv_ref[...] * (p0_ref[...] + p1_ref[...] + xwn_ref[...])
        + b_ref[...], 0.0)


def _tc_finalize(p0, p1, xwn, dinv, b):
    blk = 1000
    return pl.pallas_call(
        _fin_body,
        out_shape=jax.ShapeDtypeStruct((_N, _H), jnp.float32),
        grid=(_N // blk,),
        in_specs=[pl.BlockSpec((blk, _H), lambda i: (i, 0)),
                  pl.BlockSpec((blk, _H), lambda i: (i, 0)),
                  pl.BlockSpec((blk, _H), lambda i: (i, 0)),
                  pl.BlockSpec((blk, 1), lambda i: (i, 0)),
                  pl.BlockSpec((1, _H), lambda i: (0, 0))],
        out_specs=pl.BlockSpec((blk, _H), lambda i: (i, 0)),
    )(p0, p1, xwn, dinv, b)


_MBLK = 1000


def _blocked_proj(src_ref, wt_ref, bias, dst_ref):
    def body(i, _):
        base = pl.multiple_of(i * _MBLK, 8)
        rows = src_ref[pl.ds(base, _MBLK), :]
        dst_ref[pl.ds(base, _MBLK), :] = jnp.dot(
            rows, wt_ref[...], preferred_element_type=jnp.float32) + bias
        return 0
    lax.fori_loop(0, _N // _MBLK, body, 0)


def _gru_body(h0_ref, wih1t_ref, bih1_ref, w12_ref, w2_ref, bb_ref,
              wc_ref, bc_ref, out_ref, gi_ref, ys2_ref):
    _blocked_proj(h0_ref, wih1t_ref, bih1_ref[...], gi_ref)

    bb = bb_ref[...]

    def step(t, carry):
        h1, h2 = carry
        m12 = jnp.dot(h1.astype(jnp.bfloat16), w12_ref[...],
                      preferred_element_type=jnp.float32)
        m2 = jnp.dot(h2.astype(jnp.bfloat16), w2_ref[...],
                     preferred_element_type=jnp.float32)
        m = jnp.concatenate([m12, m2], axis=1) + bb
        gh1 = m[:, :3 * _H]
        g2 = m[:, 3 * _H:6 * _H]
        gh2 = m[:, 6 * _H:]
        tt = jnp.minimum(t, _N - 1)
        gi1 = gi_ref[pl.ds(tt, 1), :]
        r1 = jax.nn.sigmoid(gi1[:, :_H] + gh1[:, :_H])
        z1 = jax.nn.sigmoid(gi1[:, _H:2 * _H] + gh1[:, _H:2 * _H])
        n1 = jnp.tanh(gi1[:, 2 * _H:] + r1 * gh1[:, 2 * _H:])
        h1n = (1.0 - z1) * n1 + z1 * h1
        r2 = jax.nn.sigmoid(g2[:, :_H] + gh2[:, :_H])
        z2 = jax.nn.sigmoid(g2[:, _H:2 * _H] + gh2[:, _H:2 * _H])
        n2 = jnp.tanh(g2[:, 2 * _H:] + r2 * gh2[:, 2 * _H:])
        h2n = (1.0 - z2) * n2 + z2 * h2

        @pl.when(t > 0)
        def _():
            ys2_ref[pl.ds(t - 1, 1), :] = h2n

        h2n = jnp.where(t > 0, h2n, 0.0)
        return (h1n, h2n)

    h_init = jnp.zeros((1, _H), jnp.float32)
    lax.fori_loop(0, _N + 1, step, (h_init, h_init))

    bc = bc_ref[...]

    def out_body(i, _):
        base = pl.multiple_of(i * _MBLK, 8)
        rows = ys2_ref[pl.ds(base, _MBLK), :]
        out_ref[pl.ds(base, _MBLK), :] = jnp.dot(
            rows, wc_ref[...], preferred_element_type=jnp.float32) + bc
        return 0
    lax.fori_loop(0, _N // _MBLK, out_body, 0)


def _tc_gru(h0, wih1t, bih1, w12, w2, bb, wc, bc):
    return pl.pallas_call(
        _gru_body,
        out_shape=jax.ShapeDtypeStruct((_N, _O), jnp.float32),
        scratch_shapes=[
            pltpu.VMEM((_N, 3 * _H), jnp.float32),
            pltpu.VMEM((_N, _H), jnp.float32),
        ],
    )(h0, wih1t, bih1, w12, w2, bb, wc, bc)


def kernel(x, edge_index, W, b, Wih1, Whh1, bih1, bhh1,
           Wih2, Whh2, bih2, bhh2, Wc, bc):
    src = edge_index[0]
    dst = edge_index[1]

    deg_p = _sc_degree(dst)
    xw = _tc_xw(x, W)

    degp0 = deg_p[0, :_N].reshape(_N, 1)
    degp1 = deg_p[1, :_N].reshape(_N, 1)
    xwn, dinv = _tc_scale(degp0, degp1, xw)

    acc_p = _sc_message(xwn, src, dst)
    p0 = acc_p[0, :_N]
    p1 = acc_p[1, :_N]

    h0 = _tc_finalize(p0, p1, xwn, dinv, b.reshape(1, _H))
    w12 = jnp.concatenate([Whh1.T, Wih2.T], axis=1).astype(jnp.bfloat16)
    w2 = Whh2.T.astype(jnp.bfloat16)
    bb = jnp.concatenate([bhh1, bih2, bhh2]).reshape(1, 9 * _H)
    out = _tc_gru(
        h0, Wih1.T, bih1.reshape(1, 3 * _H), w12, w2, bb,
        Wc, bc.reshape(1, _O))
    return out
